# baseline (speedup 1.0000x reference)
import functools

import jax
import jax.numpy as jnp
from jax import lax
from jax.experimental import pallas as pl
from jax.experimental.pallas import tpu as pltpu
from jax.experimental.pallas import tpu_sc as plsc

VOCAB = 1000000
D = 64
BATCH = 16384
FIELDS = 26
FLAT = BATCH * FIELDS

_info = plsc.get_sparse_core_info()
NC, NS = _info.num_cores, _info.num_subcores
NW = NC * NS
B_PER_W = FLAT // NW
CHUNK = 128
NCHUNK = B_PER_W // CHUNK


def _body(idx_hbm, tab_hbm, out_hbm, idx_v, rows_v, sem):
    wid = lax.axis_index("s") * NC + lax.axis_index("c")
    base = wid * B_PER_W
    pltpu.sync_copy(idx_hbm.at[wid], idx_v)

    def chunk(j, carry):
        pltpu.async_copy(tab_hbm.at[idx_v.at[j]], rows_v, sem).wait()
        pltpu.sync_copy(rows_v, out_hbm.at[pl.ds(base + j * CHUNK, CHUNK)])
        return carry

    lax.fori_loop(0, NCHUNK, chunk, 0)


_lookup = functools.partial(
    pl.kernel,
    mesh=plsc.VectorSubcoreMesh(core_axis_name="c", subcore_axis_name="s"),
    out_type=jax.ShapeDtypeStruct((FLAT, D), jnp.float32),
    scratch_types=[
        pltpu.VMEM((NCHUNK, CHUNK), jnp.int32),
        pltpu.VMEM((CHUNK, D), jnp.float32),
        pltpu.SemaphoreType.DMA,
    ],
    compiler_params=pltpu.CompilerParams(use_tc_tiling_on_sc=False),
)(_body)


def kernel(x, embedding_table):
    idx3 = x.reshape(-1).astype(jnp.int32).reshape(NW, NCHUNK, CHUNK)
    out = _lookup(idx3, embedding_table)
    return out.reshape(BATCH, FIELDS, D)


# speedup vs baseline: 1.0246x; 1.0246x over previous
import functools

import jax
import jax.numpy as jnp
from jax import lax
from jax.experimental import pallas as pl
from jax.experimental.pallas import tpu as pltpu
from jax.experimental.pallas import tpu_sc as plsc

VOCAB = 1000000
D = 64
BATCH = 16384
FIELDS = 26
FLAT = BATCH * FIELDS

_info = plsc.get_sparse_core_info()
NC, NS = _info.num_cores, _info.num_subcores
NW = NC * NS
B_PER_W = FLAT // NW
CHUNK = 128
NCHUNK = B_PER_W // CHUNK


def _body(idx_hbm, tab_hbm, out_hbm, idx_v, rows_v, sem):
    wid = lax.axis_index("s") * NC + lax.axis_index("c")
    base = wid * B_PER_W
    pltpu.sync_copy(idx_hbm.at[wid], idx_v)

    def gather_start(j, slot):
        pltpu.async_copy(tab_hbm.at[idx_v.at[j]], rows_v.at[slot], sem.at[slot])

    def gather_wait(j, slot):
        pltpu.make_async_copy(
            tab_hbm.at[idx_v.at[j]], rows_v.at[slot], sem.at[slot]
        ).wait()

    gather_start(0, 0)

    def chunk(j, carry):
        slot = j & 1
        gather_wait(j, slot)
        gather_start(j + 1, 1 - slot)
        pltpu.sync_copy(rows_v.at[slot], out_hbm.at[pl.ds(base + j * CHUNK, CHUNK)])
        return carry

    lax.fori_loop(0, NCHUNK - 1, chunk, 0)
    last = NCHUNK - 1
    gather_wait(last, last & 1)
    pltpu.sync_copy(
        rows_v.at[last & 1], out_hbm.at[pl.ds(base + last * CHUNK, CHUNK)]
    )


_lookup = functools.partial(
    pl.kernel,
    mesh=plsc.VectorSubcoreMesh(core_axis_name="c", subcore_axis_name="s"),
    out_type=jax.ShapeDtypeStruct((FLAT, D), jnp.float32),
    scratch_types=[
        pltpu.VMEM((NCHUNK, CHUNK), jnp.int32),
        pltpu.VMEM((2, CHUNK, D), jnp.float32),
        pltpu.SemaphoreType.DMA((2,)),
    ],
    compiler_params=pltpu.CompilerParams(use_tc_tiling_on_sc=False),
)(_body)


def kernel(x, embedding_table):
    idx3 = x.reshape(-1).astype(jnp.int32).reshape(NW, NCHUNK, CHUNK)
    out = _lookup(idx3, embedding_table)
    return out.reshape(BATCH, FIELDS, D)
